# Initial kernel scaffold; baseline (speedup 1.0000x reference)
import jax
import jax.numpy as jnp
from jax.experimental import pallas as pl


def kernel(features, coords, w1, b1, g1, be1, w2, b2, g2, be2, wp, bp, gp, bep, wf, bf, wfu, bfu, gfu, befu):
    raise NotImplementedError("write your pallas kernel here")


# speedup vs baseline: 1.1295x; 1.1295x over previous
import jax
import jax.numpy as jnp
from jax import lax
from jax.experimental import pallas as pl
from jax.experimental.pallas import tpu as pltpu

R = 32
R3 = R * R * R
N = 4096
C = 64
PAD = 1088
XROWS = R3 + 2 * PAD
TILE = 1024
EPS = 1e-4
VLIM = 62 * 1024 * 1024


def _cparams(n_par):
    return pltpu.CompilerParams(
        dimension_semantics=("parallel",) * n_par,
        vmem_limit_bytes=VLIM,
    )


def _scatter_kernel(idx_ref, feat_ref, out_ref, g_ref):
    for r0 in range(0, R3, 2048):
        g_ref[r0:r0 + 2048, :] = jnp.zeros((2048, 128), jnp.float32)
    iota8 = lax.broadcasted_iota(jnp.int32, (8, 128), 0)

    U = 8

    def chunk(ci, carry):
        base = ci * U
        for u in range(U):
            n = base + u
            v = idx_ref[0, n]
            c0 = pl.multiple_of((v >> 3) << 3, 8)
            sub = v & 7
            frow = feat_ref[n]
            add = jnp.where(iota8 == sub, frow, 0.0)
            g_ref[pl.ds(c0, 8), :] = g_ref[pl.ds(c0, 8), :] + add
        return carry

    lax.fori_loop(0, N // U, chunk, 0)

    for r0 in range(0, R3, 2048):
        g = g_ref[r0:r0 + 2048, :]
        avg = g[:, 0:64] / jnp.maximum(g[:, 64:128], 1.0)
        out_ref[r0:r0 + 2048, :] = avg.astype(jnp.bfloat16)


def _make_conv_kernel(affine):
    def _conv_kernel(x_ref, w_ref, b_ref, mask_ref, sc_ref, sh_ref,
                     y_ref, st_ref, xp_ref):
        xp_ref[0:PAD, :] = jnp.zeros((PAD, 64), jnp.bfloat16)
        xp_ref[PAD + R3:XROWS, :] = jnp.zeros((PAD, 64), jnp.bfloat16)
        for r0 in range(0, R3, 2048):
            xt = x_ref[r0:r0 + 2048, :]
            if affine:
                xt = xt.astype(jnp.float32) * sc_ref[...] + sh_ref[...]
                xt = jnp.where(xt > 0, xt, 0.1 * xt)
            xp_ref[PAD + r0:PAD + r0 + 2048, :] = xt.astype(jnp.bfloat16)

        bias = b_ref[...]
        ssum = jnp.zeros((1, 64), jnp.float32)
        ssq = jnp.zeros((1, 64), jnp.float32)
        for t0 in range(0, R3, TILE):
            acc = None
            gi = 0
            for dy in (-1, 0, 1):
                for dz in (-1, 0, 1):
                    part = None
                    for dx in (-1, 0, 1):
                        s = dx * 1024 + dy * 32 + dz
                        xs = xp_ref[PAD + t0 + s:PAD + t0 + s + TILE, :]
                        wi = (dx + 1) * 9 + (dy + 1) * 3 + (dz + 1)
                        d = jnp.dot(xs, w_ref[wi],
                                    preferred_element_type=jnp.float32)
                        part = d if part is None else part + d
                    contrib = part * mask_ref[gi]
                    acc = contrib if acc is None else acc + contrib
                    gi += 1
            acc = acc + bias
            y_ref[t0:t0 + TILE, :] = acc.astype(jnp.bfloat16)
            ssum = ssum + jnp.sum(acc, axis=0, keepdims=True)
            ssq = ssq + jnp.sum(acc * acc, axis=0, keepdims=True)
        st_ref[...] = jnp.zeros((8, 128), jnp.float32)
        st_ref[0:1, 0:64] = ssum
        st_ref[1:2, 0:64] = ssq

    return _conv_kernel


def _devox_kernel(y_ref, sc_ref, sh_ref, gidx_ref, gw_ref, fz_ref,
                  vox_ref, v2_ref):
    for r0 in range(0, R3, 2048):
        v = y_ref[r0:r0 + 2048, :].astype(jnp.float32) * sc_ref[...] \
            + sh_ref[...]
        v = jnp.where(v > 0, v, 0.1 * v)
        v2_ref[r0:r0 + 2048, 0:64] = v
    v2_ref[R3:R3 + 8, :] = jnp.zeros((8, 128), jnp.float32)
    ziota = lax.broadcasted_iota(jnp.int32, (2048, 64), 0)
    for r0 in range(0, R3, 2048):
        cur = v2_ref[r0:r0 + 2048, 0:64]
        nxt = v2_ref[r0 + 1:r0 + 2049, 0:64]
        zmask = ((ziota + r0) % 32) == 31
        v2_ref[r0:r0 + 2048, 64:128] = jnp.where(zmask, cur, nxt)

    U = 8

    def chunk(ci, carry):
        base = ci * U
        for u in range(U):
            n = base + u
            acc = None
            for cc in range(4):
                i = gidx_ref[cc, n]
                c0 = pl.multiple_of((i >> 3) << 3, 8)
                ch = v2_ref[pl.ds(c0, 8), :]
                row = pltpu.roll(ch, 8 - (i & 7), axis=0)[0:1, :]
                row = row * gw_ref[cc, n]
                acc = row if acc is None else acc + row
            a = acc[:, 0:64]
            b = acc[:, 64:128]
            vox_ref[n] = a + (b - a) * fz_ref[0, n]
        return carry

    lax.fori_loop(0, N // U, chunk, 0)


def _attn_kernel(q_ref, kt_ref, fb_ref, ft_ref, wf_ref, bf_ref,
                 wp_ref, bp_ref, fz_ref, pt_ref, st_ref, p_ref):
    q = q_ref[...]
    mx = None
    for c in range(8):
        s = jnp.dot(q, kt_ref[:, c * 512:(c + 1) * 512],
                    preferred_element_type=jnp.float32) / 0.1
        cm = jnp.max(s, axis=1, keepdims=True)
        mx = cm if mx is None else jnp.maximum(mx, cm)
        p_ref[:, c * 512:(c + 1) * 512] = s
    den = None
    for c in range(8):
        p = jnp.exp(p_ref[:, c * 512:(c + 1) * 512] - mx)
        p_ref[:, c * 512:(c + 1) * 512] = p
        d = jnp.sum(p, axis=1, keepdims=True)
        den = d if den is None else den + d
    rden = 1.0 / den
    fz0 = None
    for c in range(8):
        att = p_ref[:, c * 512:(c + 1) * 512] * rden
        d = jnp.dot(att, ft_ref[c * 512:(c + 1) * 512, :],
                    preferred_element_type=jnp.float32)
        fz0 = d if fz0 is None else fz0 + d
    fz_ref[...] = jnp.dot(fz0, wf_ref[...],
                          preferred_element_type=jnp.float32) + bf_ref[...]
    ptl = jnp.dot(fb_ref[...], wp_ref[...],
                  preferred_element_type=jnp.float32) + bp_ref[...]
    pt_ref[...] = ptl
    st_ref[...] = jnp.zeros((8, 128), jnp.float32)
    st_ref[0:1, 0:64] = jnp.sum(ptl, axis=0, keepdims=True)
    st_ref[1:2, 0:64] = jnp.sum(ptl * ptl, axis=0, keepdims=True)


def _fuse_kernel(vox_ref, pt_ref, fzl_ref, ps_ref, ph_ref,
                 wv_ref, wp2_ref, wz_ref, bfu_ref, out_ref, st_ref):
    pt = pt_ref[...] * ps_ref[...] + ph_ref[...]
    pt = jnp.maximum(pt, 0.0)
    acc = jnp.dot(vox_ref[...], wv_ref[...],
                  preferred_element_type=jnp.float32)
    acc = acc + jnp.dot(pt, wp2_ref[...],
                        preferred_element_type=jnp.float32)
    acc = acc + jnp.dot(fzl_ref[...], wz_ref[...],
                        preferred_element_type=jnp.float32)
    acc = acc + bfu_ref[...]
    out_ref[...] = acc
    st_ref[...] = jnp.zeros((8, 128), jnp.float32)
    st_ref[0:1, 0:64] = jnp.sum(acc, axis=0, keepdims=True)
    st_ref[1:2, 0:64] = jnp.sum(acc * acc, axis=0, keepdims=True)


def _final_kernel(x_ref, sc_ref, sh_ref, o_ref):
    x = x_ref[...] * sc_ref[...] + sh_ref[...]
    o_ref[...] = jnp.maximum(x, 0.0)


def _finalize_stats(st, count, g, be):
    tot = jnp.sum(st.reshape(-1, 8, 128), axis=0)
    m = tot[0, 0:64] / count
    sq = tot[1, 0:64] / count
    var = sq - m * m
    sc = g * lax.rsqrt(var + EPS)
    sh = be - m * sc
    return sc.reshape(1, 64), sh.reshape(1, 64)


def kernel(features, coords, w1, b1, g1, be1, w2, b2, g2, be2,
           wp, bp, gp, bep, wf, bf, wfu, bfu, gfu, befu):
    B = features.shape[0]
    f32 = jnp.float32

    nc = coords - coords.mean(axis=2, keepdims=True)
    norm = jnp.linalg.norm(nc, axis=1, keepdims=True)
    nc = nc / (norm.max(axis=2, keepdims=True) * 2.0) + 0.5
    nc = jnp.clip(nc * R, 0.0, R - 1)
    vox = jnp.round(nc).astype(jnp.int32)
    sidx = ((vox[:, 0] * R + vox[:, 1]) * R + vox[:, 2]).reshape(B, 1, N)

    lo_f = jnp.floor(nc)
    fr = nc - lo_f
    lo = lo_f.astype(jnp.int32)
    hi = jnp.minimum(lo + 1, R - 1)
    lox, loy, loz = lo[:, 0], lo[:, 1], lo[:, 2]
    hix, hiy = hi[:, 0], hi[:, 1]
    fx, fy, fzf = fr[:, 0], fr[:, 1], fr[:, 2]
    gidx = jnp.stack([(lox * R + loy) * R + loz,
                      (lox * R + hiy) * R + loz,
                      (hix * R + loy) * R + loz,
                      (hix * R + hiy) * R + loz], axis=1)
    gw = jnp.stack([(1 - fx) * (1 - fy), (1 - fx) * fy,
                    fx * (1 - fy), fx * fy], axis=1)
    fzp = fzf.reshape(B, 1, N)

    featT = features.transpose(0, 2, 1)
    featpad = jnp.concatenate(
        [featT, jnp.ones((B, N, 64), f32)], axis=2).reshape(B, N, 1, 128)

    avg = pl.pallas_call(
        _scatter_kernel,
        grid=(B,),
        in_specs=[
            pl.BlockSpec((None, 1, N), lambda b: (b, 0, 0),
                         memory_space=pltpu.SMEM),
            pl.BlockSpec((None, N, 1, 128), lambda b: (b, 0, 0, 0)),
        ],
        out_specs=pl.BlockSpec((None, R3, 64), lambda b: (b, 0, 0)),
        out_shape=jax.ShapeDtypeStruct((B, R3, 64), jnp.bfloat16),
        scratch_shapes=[pltpu.VMEM((R3, 128), f32)],
        compiler_params=_cparams(1),
    )(sidx, featpad)

    wm1 = w1.transpose(2, 3, 4, 1, 0).reshape(27, C, C).astype(jnp.bfloat16)
    wm2 = w2.transpose(2, 3, 4, 1, 0).reshape(27, C, C).astype(jnp.bfloat16)
    rr = jnp.arange(TILE)
    yy = (rr // 32) % 32
    zz = rr % 32
    masks = []
    for dy in (-1, 0, 1):
        for dz in (-1, 0, 1):
            ok = ((yy + dy >= 0) & (yy + dy < 32)
                  & (zz + dz >= 0) & (zz + dz < 32)).astype(f32)
            masks.append(jnp.broadcast_to(ok[:, None], (TILE, 64)))
    mask_arr = jnp.stack(masks)

    def conv_call(x, wmat, bias, scale, shift, affine):
        return pl.pallas_call(
            _make_conv_kernel(affine),
            grid=(B,),
            in_specs=[
                pl.BlockSpec((None, R3, 64), lambda b: (b, 0, 0)),
                pl.BlockSpec((27, C, C), lambda b: (0, 0, 0)),
                pl.BlockSpec((1, 64), lambda b: (0, 0)),
                pl.BlockSpec((9, TILE, 64), lambda b: (0, 0, 0)),
                pl.BlockSpec((1, 64), lambda b: (0, 0)),
                pl.BlockSpec((1, 64), lambda b: (0, 0)),
            ],
            out_specs=[
                pl.BlockSpec((None, R3, 64), lambda b: (b, 0, 0)),
                pl.BlockSpec((None, 8, 128), lambda b: (b, 0, 0)),
            ],
            out_shape=[
                jax.ShapeDtypeStruct((B, R3, 64), jnp.bfloat16),
                jax.ShapeDtypeStruct((B, 8, 128), f32),
            ],
            scratch_shapes=[pltpu.VMEM((XROWS, 64), jnp.bfloat16)],
            compiler_params=_cparams(1),
        )(x, wmat, bias.reshape(1, 64), mask_arr, scale, shift)

    one = jnp.ones((1, 64), f32)
    zero = jnp.zeros((1, 64), f32)
    y1, st1 = conv_call(avg, wm1, b1, one, zero, affine=False)
    sc1, sh1 = _finalize_stats(st1, float(B * R3), g1, be1)
    y2, st2 = conv_call(y1, wm2, b2, sc1, sh1, affine=True)
    sc2, sh2 = _finalize_stats(st2, float(B * R3), g2, be2)

    vox4 = pl.pallas_call(
        _devox_kernel,
        grid=(B,),
        in_specs=[
            pl.BlockSpec((None, R3, 64), lambda b: (b, 0, 0)),
            pl.BlockSpec((1, 64), lambda b: (0, 0)),
            pl.BlockSpec((1, 64), lambda b: (0, 0)),
            pl.BlockSpec((None, 4, N), lambda b: (b, 0, 0),
                         memory_space=pltpu.SMEM),
            pl.BlockSpec((None, 4, N), lambda b: (b, 0, 0),
                         memory_space=pltpu.SMEM),
            pl.BlockSpec((None, 1, N), lambda b: (b, 0, 0),
                         memory_space=pltpu.SMEM),
        ],
        out_specs=pl.BlockSpec((None, N, 1, 64), lambda b: (b, 0, 0, 0)),
        out_shape=jax.ShapeDtypeStruct((B, N, 1, 64), f32),
        scratch_shapes=[pltpu.VMEM((R3 + 8, 128), f32)],
        compiler_params=_cparams(1),
    )(y2, sc2, sh2, gidx, gw, fzp)
    voxf = vox4.reshape(B, N, 64)

    cn = coords.transpose(0, 2, 1)
    cnn = cn / jnp.maximum(jnp.linalg.norm(cn, axis=-1, keepdims=True),
                           1e-12)
    cnp = jnp.pad(cnn, ((0, 0), (0, 0), (0, 5)))
    cnpT = cnp.transpose(0, 2, 1)

    RB = 8
    BQ = N // RB
    fzl, ptl, stp = pl.pallas_call(
        _attn_kernel,
        grid=(B, RB),
        in_specs=[
            pl.BlockSpec((None, BQ, 8), lambda b, r: (b, r, 0)),
            pl.BlockSpec((None, 8, N), lambda b, r: (b, 0, 0)),
            pl.BlockSpec((None, BQ, 64), lambda b, r: (b, r, 0)),
            pl.BlockSpec((None, N, 64), lambda b, r: (b, 0, 0)),
            pl.BlockSpec((64, 64), lambda b, r: (0, 0)),
            pl.BlockSpec((1, 64), lambda b, r: (0, 0)),
            pl.BlockSpec((64, 64), lambda b, r: (0, 0)),
            pl.BlockSpec((1, 64), lambda b, r: (0, 0)),
        ],
        out_specs=[
            pl.BlockSpec((None, BQ, 64), lambda b, r: (b, r, 0)),
            pl.BlockSpec((None, BQ, 64), lambda b, r: (b, r, 0)),
            pl.BlockSpec((None, None, 8, 128), lambda b, r: (b, r, 0, 0)),
        ],
        out_shape=[
            jax.ShapeDtypeStruct((B, N, 64), f32),
            jax.ShapeDtypeStruct((B, N, 64), f32),
            jax.ShapeDtypeStruct((B, RB, 8, 128), f32),
        ],
        scratch_shapes=[pltpu.VMEM((BQ, N), f32)],
        compiler_params=_cparams(2),
    )(cnp, cnpT, featT, featT, wf.T, bf.reshape(1, 64),
      wp.T, bp.reshape(1, 64))
    scp, shp = _finalize_stats(stp, float(B * N), gp, bep)

    wv = wfu[:, 0:64].T
    wp2 = wfu[:, 64:128].T
    wz = wfu[:, 128:192].T
    fusedl, stf = pl.pallas_call(
        _fuse_kernel,
        grid=(B, RB),
        in_specs=[
            pl.BlockSpec((None, BQ, 64), lambda b, r: (b, r, 0)),
            pl.BlockSpec((None, BQ, 64), lambda b, r: (b, r, 0)),
            pl.BlockSpec((None, BQ, 64), lambda b, r: (b, r, 0)),
            pl.BlockSpec((1, 64), lambda b, r: (0, 0)),
            pl.BlockSpec((1, 64), lambda b, r: (0, 0)),
            pl.BlockSpec((64, 64), lambda b, r: (0, 0)),
            pl.BlockSpec((64, 64), lambda b, r: (0, 0)),
            pl.BlockSpec((64, 64), lambda b, r: (0, 0)),
            pl.BlockSpec((1, 64), lambda b, r: (0, 0)),
        ],
        out_specs=[
            pl.BlockSpec((None, BQ, 64), lambda b, r: (b, r, 0)),
            pl.BlockSpec((None, None, 8, 128), lambda b, r: (b, r, 0, 0)),
        ],
        out_shape=[
            jax.ShapeDtypeStruct((B, N, 64), f32),
            jax.ShapeDtypeStruct((B, RB, 8, 128), f32),
        ],
        compiler_params=_cparams(2),
    )(voxf, ptl, fzl, scp, shp, wv, wp2, wz, bfu.reshape(1, 64))
    scf, shf = _finalize_stats(stf, float(B * N), gfu, befu)

    out = pl.pallas_call(
        _final_kernel,
        grid=(B,),
        in_specs=[
            pl.BlockSpec((None, N, 64), lambda b: (b, 0, 0)),
            pl.BlockSpec((1, 64), lambda b: (0, 0)),
            pl.BlockSpec((1, 64), lambda b: (0, 0)),
        ],
        out_specs=pl.BlockSpec((None, N, 64), lambda b: (b, 0, 0)),
        out_shape=jax.ShapeDtypeStruct((B, N, 64), f32),
        compiler_params=_cparams(1),
    )(fusedl, scf, shf)

    fused = out.transpose(0, 2, 1)
    return (fused, coords)


# speedup vs baseline: 1.1795x; 1.0442x over previous
import jax
import jax.numpy as jnp
from jax import lax
from jax.experimental import pallas as pl
from jax.experimental.pallas import tpu as pltpu

R = 32
R3 = R * R * R
N = 4096
C = 64
PAD = 1088
XROWS = R3 + 2 * PAD
TILE = 1024
EPS = 1e-4
VLIM = 62 * 1024 * 1024


def _cparams(n_par):
    return pltpu.CompilerParams(
        dimension_semantics=("parallel",) * n_par,
        vmem_limit_bytes=VLIM,
    )


def _scatter_kernel(idx_ref, feat_ref, out_ref, ga_ref, gb_ref):
    for r0 in range(0, R3, 2048):
        ga_ref[r0:r0 + 2048, :] = jnp.zeros((2048, 128), jnp.float32)
        gb_ref[r0:r0 + 2048, :] = jnp.zeros((2048, 128), jnp.float32)
    iota8 = lax.broadcasted_iota(jnp.int32, (8, 128), 0)

    U = 8

    def chunk(ci, carry):
        base = ci * U
        for u in range(U):
            n = base + u
            v = idx_ref[0, n]
            c0 = pl.multiple_of((v >> 3) << 3, 8)
            sub = v & 7
            frow = feat_ref[n]
            add = jnp.where(iota8 == sub, frow, 0.0)
            g_ref = ga_ref if (u % 2 == 0) else gb_ref
            g_ref[pl.ds(c0, 8), :] = g_ref[pl.ds(c0, 8), :] + add
        return carry

    lax.fori_loop(0, N // U, chunk, 0)

    for r0 in range(0, R3, 2048):
        g = ga_ref[r0:r0 + 2048, :] + gb_ref[r0:r0 + 2048, :]
        avg = g[:, 0:64] / jnp.maximum(g[:, 64:128], 1.0)
        out_ref[r0:r0 + 2048, :] = avg.astype(jnp.bfloat16)


def _make_conv_kernel(affine):
    def _conv_kernel(x_ref, w_ref, b_ref, mask_ref, sc_ref, sh_ref,
                     y_ref, st_ref, xp_ref):
        xp_ref[0:PAD, :] = jnp.zeros((PAD, 64), jnp.float32)
        xp_ref[PAD + R3:XROWS, :] = jnp.zeros((PAD, 64), jnp.float32)
        for r0 in range(0, R3, 2048):
            xt = x_ref[r0:r0 + 2048, :].astype(jnp.float32)
            if affine:
                xt = xt * sc_ref[...] + sh_ref[...]
                xt = jnp.where(xt > 0, xt, 0.1 * xt)
            xp_ref[PAD + r0:PAD + r0 + 2048, :] = xt

        bias = b_ref[...]
        ssum = jnp.zeros((1, 64), jnp.float32)
        ssq = jnp.zeros((1, 64), jnp.float32)
        for t0 in range(0, R3, TILE):
            acc = None
            gi = 0
            for dy in (-1, 0, 1):
                for dz in (-1, 0, 1):
                    part = None
                    for dx in (-1, 0, 1):
                        s = dx * 1024 + dy * 32 + dz
                        xs = xp_ref[PAD + t0 + s:PAD + t0 + s + TILE, :]
                        wi = (dx + 1) * 9 + (dy + 1) * 3 + (dz + 1)
                        d = jnp.dot(xs, w_ref[wi],
                                    preferred_element_type=jnp.float32)
                        part = d if part is None else part + d
                    contrib = part * mask_ref[gi]
                    acc = contrib if acc is None else acc + contrib
                    gi += 1
            acc = acc + bias
            y_ref[t0:t0 + TILE, :] = acc.astype(jnp.bfloat16)
            ssum = ssum + jnp.sum(acc, axis=0, keepdims=True)
            ssq = ssq + jnp.sum(acc * acc, axis=0, keepdims=True)
        st_ref[...] = jnp.zeros((8, 128), jnp.float32)
        st_ref[0:1, 0:64] = ssum
        st_ref[1:2, 0:64] = ssq

    return _conv_kernel


def _devox_kernel(y_ref, sc_ref, sh_ref, gidx_ref, gw_ref, fz_ref,
                  vox_ref, v2_ref):
    for r0 in range(0, R3, 2048):
        v = y_ref[r0:r0 + 2048, :].astype(jnp.float32) * sc_ref[...] \
            + sh_ref[...]
        v = jnp.where(v > 0, v, 0.1 * v)
        v2_ref[r0:r0 + 2048, 0:64] = v
    v2_ref[R3:R3 + 8, :] = jnp.zeros((8, 128), jnp.float32)
    ziota = lax.broadcasted_iota(jnp.int32, (2048, 64), 0)
    for r0 in range(0, R3, 2048):
        cur = v2_ref[r0:r0 + 2048, 0:64]
        nxt = v2_ref[r0 + 1:r0 + 2049, 0:64]
        zmask = ((ziota + r0) % 32) == 31
        v2_ref[r0:r0 + 2048, 64:128] = jnp.where(zmask, cur, nxt)

    U = 8

    def chunk(ci, carry):
        base = ci * U
        for u in range(U):
            n = base + u
            acc = None
            for cc in range(4):
                i = gidx_ref[cc, n]
                c0 = pl.multiple_of((i >> 3) << 3, 8)
                ch = v2_ref[pl.ds(c0, 8), :]
                row = pltpu.roll(ch, 8 - (i & 7), axis=0)[0:1, :]
                row = row * gw_ref[cc, n]
                acc = row if acc is None else acc + row
            a = acc[:, 0:64]
            b = acc[:, 64:128]
            vox_ref[n] = a + (b - a) * fz_ref[0, n]
        return carry

    lax.fori_loop(0, N // U, chunk, 0)


def _attn_kernel(q_ref, kt_ref, fb_ref, ft_ref, wf_ref, bf_ref,
                 wp_ref, bp_ref, fz_ref, pt_ref, st_ref, p_ref):
    q = q_ref[...]
    mx = None
    for c in range(8):
        s = jnp.dot(q, kt_ref[:, c * 512:(c + 1) * 512],
                    preferred_element_type=jnp.float32) / 0.1
        cm = jnp.max(s, axis=1, keepdims=True)
        mx = cm if mx is None else jnp.maximum(mx, cm)
        p_ref[:, c * 512:(c + 1) * 512] = s
    den = None
    for c in range(8):
        p = jnp.exp(p_ref[:, c * 512:(c + 1) * 512] - mx)
        p_ref[:, c * 512:(c + 1) * 512] = p
        d = jnp.sum(p, axis=1, keepdims=True)
        den = d if den is None else den + d
    rden = 1.0 / den
    fz0 = None
    for c in range(8):
        att = p_ref[:, c * 512:(c + 1) * 512] * rden
        d = jnp.dot(att, ft_ref[c * 512:(c + 1) * 512, :],
                    preferred_element_type=jnp.float32)
        fz0 = d if fz0 is None else fz0 + d
    fz_ref[...] = jnp.dot(fz0, wf_ref[...],
                          preferred_element_type=jnp.float32) + bf_ref[...]
    ptl = jnp.dot(fb_ref[...], wp_ref[...],
                  preferred_element_type=jnp.float32) + bp_ref[...]
    pt_ref[...] = ptl
    st_ref[...] = jnp.zeros((8, 128), jnp.float32)
    st_ref[0:1, 0:64] = jnp.sum(ptl, axis=0, keepdims=True)
    st_ref[1:2, 0:64] = jnp.sum(ptl * ptl, axis=0, keepdims=True)


def _fuse_kernel(vox_ref, pt_ref, fzl_ref, ps_ref, ph_ref,
                 wv_ref, wp2_ref, wz_ref, bfu_ref, out_ref, st_ref):
    pt = pt_ref[...] * ps_ref[...] + ph_ref[...]
    pt = jnp.maximum(pt, 0.0)
    acc = jnp.dot(vox_ref[...], wv_ref[...],
                  preferred_element_type=jnp.float32)
    acc = acc + jnp.dot(pt, wp2_ref[...],
                        preferred_element_type=jnp.float32)
    acc = acc + jnp.dot(fzl_ref[...], wz_ref[...],
                        preferred_element_type=jnp.float32)
    acc = acc + bfu_ref[...]
    out_ref[...] = acc
    st_ref[...] = jnp.zeros((8, 128), jnp.float32)
    st_ref[0:1, 0:64] = jnp.sum(acc, axis=0, keepdims=True)
    st_ref[1:2, 0:64] = jnp.sum(acc * acc, axis=0, keepdims=True)


def _final_kernel(x_ref, sc_ref, sh_ref, o_ref):
    x = x_ref[...] * sc_ref[...] + sh_ref[...]
    o_ref[...] = jnp.maximum(x, 0.0)


def _finalize_stats(st, count, g, be):
    tot = jnp.sum(st.reshape(-1, 8, 128), axis=0)
    m = tot[0, 0:64] / count
    sq = tot[1, 0:64] / count
    var = sq - m * m
    sc = g * lax.rsqrt(var + EPS)
    sh = be - m * sc
    return sc.reshape(1, 64), sh.reshape(1, 64)


def kernel(features, coords, w1, b1, g1, be1, w2, b2, g2, be2,
           wp, bp, gp, bep, wf, bf, wfu, bfu, gfu, befu):
    B = features.shape[0]
    f32 = jnp.float32

    nc = coords - coords.mean(axis=2, keepdims=True)
    norm = jnp.linalg.norm(nc, axis=1, keepdims=True)
    nc = nc / (norm.max(axis=2, keepdims=True) * 2.0) + 0.5
    nc = jnp.clip(nc * R, 0.0, R - 1)
    vox = jnp.round(nc).astype(jnp.int32)
    sidx = ((vox[:, 0] * R + vox[:, 1]) * R + vox[:, 2]).reshape(B, 1, N)

    lo_f = jnp.floor(nc)
    fr = nc - lo_f
    lo = lo_f.astype(jnp.int32)
    hi = jnp.minimum(lo + 1, R - 1)
    lox, loy, loz = lo[:, 0], lo[:, 1], lo[:, 2]
    hix, hiy = hi[:, 0], hi[:, 1]
    fx, fy, fzf = fr[:, 0], fr[:, 1], fr[:, 2]
    gidx = jnp.stack([(lox * R + loy) * R + loz,
                      (lox * R + hiy) * R + loz,
                      (hix * R + loy) * R + loz,
                      (hix * R + hiy) * R + loz], axis=1)
    gw = jnp.stack([(1 - fx) * (1 - fy), (1 - fx) * fy,
                    fx * (1 - fy), fx * fy], axis=1)
    fzp = fzf.reshape(B, 1, N)

    featT = features.transpose(0, 2, 1)
    featpad = jnp.concatenate(
        [featT, jnp.ones((B, N, 64), f32)], axis=2).reshape(B, N, 1, 128)

    avg = pl.pallas_call(
        _scatter_kernel,
        grid=(B,),
        in_specs=[
            pl.BlockSpec((None, 1, N), lambda b: (b, 0, 0),
                         memory_space=pltpu.SMEM),
            pl.BlockSpec((None, N, 1, 128), lambda b: (b, 0, 0, 0)),
        ],
        out_specs=pl.BlockSpec((None, R3, 64), lambda b: (b, 0, 0)),
        out_shape=jax.ShapeDtypeStruct((B, R3, 64), jnp.bfloat16),
        scratch_shapes=[pltpu.VMEM((R3, 128), f32),
                        pltpu.VMEM((R3, 128), f32)],
        compiler_params=_cparams(1),
    )(sidx, featpad)

    wm1 = w1.transpose(2, 3, 4, 1, 0).reshape(27, C, C).astype(jnp.bfloat16)
    wm2 = w2.transpose(2, 3, 4, 1, 0).reshape(27, C, C).astype(jnp.bfloat16)
    rr = jnp.arange(TILE)
    yy = (rr // 32) % 32
    zz = rr % 32
    masks = []
    for dy in (-1, 0, 1):
        for dz in (-1, 0, 1):
            ok = ((yy + dy >= 0) & (yy + dy < 32)
                  & (zz + dz >= 0) & (zz + dz < 32)).astype(f32)
            masks.append(jnp.broadcast_to(ok[:, None], (TILE, 64)))
    mask_arr = jnp.stack(masks)

    def conv_call(x, wmat, bias, scale, shift, affine):
        return pl.pallas_call(
            _make_conv_kernel(affine),
            grid=(B,),
            in_specs=[
                pl.BlockSpec((None, R3, 64), lambda b: (b, 0, 0)),
                pl.BlockSpec((27, C, C), lambda b: (0, 0, 0)),
                pl.BlockSpec((1, 64), lambda b: (0, 0)),
                pl.BlockSpec((9, TILE, 64), lambda b: (0, 0, 0)),
                pl.BlockSpec((1, 64), lambda b: (0, 0)),
                pl.BlockSpec((1, 64), lambda b: (0, 0)),
            ],
            out_specs=[
                pl.BlockSpec((None, R3, 64), lambda b: (b, 0, 0)),
                pl.BlockSpec((None, 8, 128), lambda b: (b, 0, 0)),
            ],
            out_shape=[
                jax.ShapeDtypeStruct((B, R3, 64), jnp.bfloat16),
                jax.ShapeDtypeStruct((B, 8, 128), f32),
            ],
            scratch_shapes=[pltpu.VMEM((XROWS, 64), jnp.float32)],
            compiler_params=_cparams(1),
        )(x, wmat, bias.reshape(1, 64), mask_arr, scale, shift)

    one = jnp.ones((1, 64), f32)
    zero = jnp.zeros((1, 64), f32)
    y1, st1 = conv_call(avg, wm1, b1, one, zero, affine=False)
    sc1, sh1 = _finalize_stats(st1, float(B * R3), g1, be1)
    y2, st2 = conv_call(y1, wm2, b2, sc1, sh1, affine=True)
    sc2, sh2 = _finalize_stats(st2, float(B * R3), g2, be2)

    vox4 = pl.pallas_call(
        _devox_kernel,
        grid=(B,),
        in_specs=[
            pl.BlockSpec((None, R3, 64), lambda b: (b, 0, 0)),
            pl.BlockSpec((1, 64), lambda b: (0, 0)),
            pl.BlockSpec((1, 64), lambda b: (0, 0)),
            pl.BlockSpec((None, 4, N), lambda b: (b, 0, 0),
                         memory_space=pltpu.SMEM),
            pl.BlockSpec((None, 4, N), lambda b: (b, 0, 0),
                         memory_space=pltpu.SMEM),
            pl.BlockSpec((None, 1, N), lambda b: (b, 0, 0),
                         memory_space=pltpu.SMEM),
        ],
        out_specs=pl.BlockSpec((None, N, 1, 64), lambda b: (b, 0, 0, 0)),
        out_shape=jax.ShapeDtypeStruct((B, N, 1, 64), f32),
        scratch_shapes=[pltpu.VMEM((R3 + 8, 128), f32)],
        compiler_params=_cparams(1),
    )(y2, sc2, sh2, gidx, gw, fzp)
    voxf = vox4.reshape(B, N, 64)

    cn = coords.transpose(0, 2, 1)
    cnn = cn / jnp.maximum(jnp.linalg.norm(cn, axis=-1, keepdims=True),
                           1e-12)
    cnp = jnp.pad(cnn, ((0, 0), (0, 0), (0, 5)))
    cnpT = cnp.transpose(0, 2, 1)

    RB = 8
    BQ = N // RB
    fzl, ptl, stp = pl.pallas_call(
        _attn_kernel,
        grid=(B, RB),
        in_specs=[
            pl.BlockSpec((None, BQ, 8), lambda b, r: (b, r, 0)),
            pl.BlockSpec((None, 8, N), lambda b, r: (b, 0, 0)),
            pl.BlockSpec((None, BQ, 64), lambda b, r: (b, r, 0)),
            pl.BlockSpec((None, N, 64), lambda b, r: (b, 0, 0)),
            pl.BlockSpec((64, 64), lambda b, r: (0, 0)),
            pl.BlockSpec((1, 64), lambda b, r: (0, 0)),
            pl.BlockSpec((64, 64), lambda b, r: (0, 0)),
            pl.BlockSpec((1, 64), lambda b, r: (0, 0)),
        ],
        out_specs=[
            pl.BlockSpec((None, BQ, 64), lambda b, r: (b, r, 0)),
            pl.BlockSpec((None, BQ, 64), lambda b, r: (b, r, 0)),
            pl.BlockSpec((None, None, 8, 128), lambda b, r: (b, r, 0, 0)),
        ],
        out_shape=[
            jax.ShapeDtypeStruct((B, N, 64), f32),
            jax.ShapeDtypeStruct((B, N, 64), f32),
            jax.ShapeDtypeStruct((B, RB, 8, 128), f32),
        ],
        scratch_shapes=[pltpu.VMEM((BQ, N), f32)],
        compiler_params=_cparams(2),
    )(cnp, cnpT, featT, featT, wf.T, bf.reshape(1, 64),
      wp.T, bp.reshape(1, 64))
    scp, shp = _finalize_stats(stp, float(B * N), gp, bep)

    wv = wfu[:, 0:64].T
    wp2 = wfu[:, 64:128].T
    wz = wfu[:, 128:192].T
    fusedl, stf = pl.pallas_call(
        _fuse_kernel,
        grid=(B, RB),
        in_specs=[
            pl.BlockSpec((None, BQ, 64), lambda b, r: (b, r, 0)),
            pl.BlockSpec((None, BQ, 64), lambda b, r: (b, r, 0)),
            pl.BlockSpec((None, BQ, 64), lambda b, r: (b, r, 0)),
            pl.BlockSpec((1, 64), lambda b, r: (0, 0)),
            pl.BlockSpec((1, 64), lambda b, r: (0, 0)),
            pl.BlockSpec((64, 64), lambda b, r: (0, 0)),
            pl.BlockSpec((64, 64), lambda b, r: (0, 0)),
            pl.BlockSpec((64, 64), lambda b, r: (0, 0)),
            pl.BlockSpec((1, 64), lambda b, r: (0, 0)),
        ],
        out_specs=[
            pl.BlockSpec((None, BQ, 64), lambda b, r: (b, r, 0)),
            pl.BlockSpec((None, None, 8, 128), lambda b, r: (b, r, 0, 0)),
        ],
        out_shape=[
            jax.ShapeDtypeStruct((B, N, 64), f32),
            jax.ShapeDtypeStruct((B, RB, 8, 128), f32),
        ],
        compiler_params=_cparams(2),
    )(voxf, ptl, fzl, scp, shp, wv, wp2, wz, bfu.reshape(1, 64))
    scf, shf = _finalize_stats(stf, float(B * N), gfu, befu)

    out = pl.pallas_call(
        _final_kernel,
        grid=(B,),
        in_specs=[
            pl.BlockSpec((None, N, 64), lambda b: (b, 0, 0)),
            pl.BlockSpec((1, 64), lambda b: (0, 0)),
            pl.BlockSpec((1, 64), lambda b: (0, 0)),
        ],
        out_specs=pl.BlockSpec((None, N, 64), lambda b: (b, 0, 0)),
        out_shape=jax.ShapeDtypeStruct((B, N, 64), f32),
        compiler_params=_cparams(1),
    )(fusedl, scf, shf)

    fused = out.transpose(0, 2, 1)
    return (fused, coords)
